# baseline (speedup 1.0000x reference)
import jax
import jax.numpy as jnp
from jax.experimental import pallas as pl
from jax.experimental.pallas import tpu as pltpu

N = 100000
D = 512
K = 5000
BR = 2000


def _score_kernel(a_ref, w_ref, o_ref):
    w = w_ref[...]
    norm = jnp.maximum(jnp.sqrt(jnp.sum(w * w)), 1e-6)
    s = jnp.dot(a_ref[...], w, preferred_element_type=jnp.float32)
    o_ref[...] = s / norm


def _gather_kernel(idx_ref, val_ref, row_ref, o_ref):
    i = pl.program_id(0)
    o_ref[...] = row_ref[...] * jnp.tanh(val_ref[i])


def kernel(node_embs, scorer):
    scores = pl.pallas_call(
        _score_kernel,
        grid=(N // BR,),
        in_specs=[
            pl.BlockSpec((BR, D), lambda i: (i, 0)),
            pl.BlockSpec((D, 1), lambda i: (0, 0)),
        ],
        out_specs=pl.BlockSpec((BR, 1), lambda i: (i, 0)),
        out_shape=jax.ShapeDtypeStruct((N, 1), jnp.float32),
    )(node_embs, scorer).reshape(-1)

    vals, idx = jax.lax.top_k(scores, K)

    gathered = pl.pallas_call(
        _gather_kernel,
        grid_spec=pltpu.PrefetchScalarGridSpec(
            num_scalar_prefetch=2,
            grid=(K,),
            in_specs=[
                pl.BlockSpec((1, 1, D), lambda i, idx_ref, val_ref: (idx_ref[i], 0, 0)),
            ],
            out_specs=pl.BlockSpec((1, 1, D), lambda i, idx_ref, val_ref: (i, 0, 0)),
        ),
        out_shape=jax.ShapeDtypeStruct((K, 1, D), jnp.float32),
    )(idx, vals, node_embs.reshape(N, 1, D))

    return gathered.reshape(K, D).T


# speedup vs baseline: 7.2893x; 7.2893x over previous
import jax
import jax.numpy as jnp
from jax.experimental import pallas as pl
from jax.experimental.pallas import tpu as pltpu

N = 100000
D = 512
K = 5000
BR = 2000


def _score_kernel(a_ref, w_ref, o_ref):
    w = w_ref[...]
    norm = jnp.maximum(jnp.sqrt(jnp.sum(w * w)), 1e-6)
    s = jnp.dot(a_ref[...], w, preferred_element_type=jnp.float32)
    o_ref[...] = s / norm


BG = 1000


def _gather_kernel(idx_ref, val_ref, hbm_ref, o_ref, scratch, sems):
    i = pl.program_id(0)
    base = i * BG

    def issue(r, c):
        g = idx_ref[base + r]
        pltpu.make_async_copy(hbm_ref.at[g], scratch.at[r], sems.at[r]).start()
        return c

    jax.lax.fori_loop(0, BG, issue, 0)

    def wait(r, c):
        g = idx_ref[base + r]
        pltpu.make_async_copy(hbm_ref.at[g], scratch.at[r], sems.at[r]).wait()
        return c

    jax.lax.fori_loop(0, BG, wait, 0)
    o_ref[...] = scratch[...] * jnp.tanh(val_ref[0])


def kernel(node_embs, scorer):
    scores = pl.pallas_call(
        _score_kernel,
        grid=(N // BR,),
        in_specs=[
            pl.BlockSpec((BR, D), lambda i: (i, 0)),
            pl.BlockSpec((D, 1), lambda i: (0, 0)),
        ],
        out_specs=pl.BlockSpec((BR, 1), lambda i: (i, 0)),
        out_shape=jax.ShapeDtypeStruct((N, 1), jnp.float32),
    )(node_embs, scorer).reshape(-1)

    vals, idx = jax.lax.top_k(scores, K)

    gathered = pl.pallas_call(
        _gather_kernel,
        grid_spec=pltpu.PrefetchScalarGridSpec(
            num_scalar_prefetch=1,
            grid=(K // BG,),
            in_specs=[
                pl.BlockSpec((1, BG, 1), lambda i, idx_ref: (i, 0, 0)),
                pl.BlockSpec(memory_space=pltpu.MemorySpace.HBM),
            ],
            out_specs=pl.BlockSpec((BG, D), lambda i, idx_ref: (i, 0)),
            scratch_shapes=[
                pltpu.VMEM((BG, D), jnp.float32),
                pltpu.SemaphoreType.DMA((BG,)),
            ],
        ),
        out_shape=jax.ShapeDtypeStruct((K, D), jnp.float32),
    )(idx, vals.reshape(K // BG, BG, 1), node_embs)

    return gathered.T


# speedup vs baseline: 8.6278x; 1.1836x over previous
import functools

import jax
import jax.numpy as jnp
from jax import lax
from jax.experimental import pallas as pl
from jax.experimental.pallas import tpu as pltpu
from jax.experimental.pallas import tpu_sc as plsc

N = 100000
D = 512
K = 5000
BR = 2000
KP = 5120
BS = 1000

_INFO = plsc.get_sparse_core_info()
_NW = _INFO.num_cores * _INFO.num_subcores
_BPW = KP // _NW


def _score_kernel(a_ref, w_ref, o_ref):
    w = w_ref[...]
    norm = jnp.maximum(jnp.sqrt(jnp.sum(w * w)), 1e-6)
    s = jnp.dot(a_ref[...], w, preferred_element_type=jnp.float32)
    o_ref[...] = s / norm


def _sc_gather(table_hbm, idx_hbm, out_hbm, idx_v, rows_v, sem):
    wid = lax.axis_index("s") * _INFO.num_cores + lax.axis_index("c")
    base = wid * _BPW
    pltpu.sync_copy(idx_hbm.at[pl.ds(base, _BPW)], idx_v)
    pltpu.async_copy(table_hbm.at[idx_v], rows_v, sem).wait()
    pltpu.sync_copy(rows_v, out_hbm.at[pl.ds(base, _BPW)])


_gather_call = pl.kernel(
    _sc_gather,
    mesh=plsc.VectorSubcoreMesh(core_axis_name="c", subcore_axis_name="s"),
    out_type=jax.ShapeDtypeStruct((KP, D), jnp.float32),
    scratch_types=[
        pltpu.VMEM((_BPW,), jnp.int32),
        pltpu.VMEM((_BPW, D), jnp.float32),
        pltpu.SemaphoreType.DMA,
    ],
)


def _scale_kernel(val_ref, g_ref, o_ref):
    o_ref[...] = g_ref[...] * jnp.tanh(val_ref[0])


def kernel(node_embs, scorer):
    scores = pl.pallas_call(
        _score_kernel,
        grid=(N // BR,),
        in_specs=[
            pl.BlockSpec((BR, D), lambda i: (i, 0)),
            pl.BlockSpec((D, 1), lambda i: (0, 0)),
        ],
        out_specs=pl.BlockSpec((BR, 1), lambda i: (i, 0)),
        out_shape=jax.ShapeDtypeStruct((N, 1), jnp.float32),
    )(node_embs, scorer).reshape(-1)

    vals, idx = jax.lax.top_k(scores, K)

    idx_p = jnp.concatenate([idx, jnp.zeros((KP - K,), jnp.int32)])
    gathered = _gather_call(node_embs, idx_p)

    scaled = pl.pallas_call(
        _scale_kernel,
        grid=(K // BS,),
        in_specs=[
            pl.BlockSpec((1, BS, 1), lambda i: (i, 0, 0)),
            pl.BlockSpec((BS, D), lambda i: (i, 0)),
        ],
        out_specs=pl.BlockSpec((BS, D), lambda i: (i, 0)),
        out_shape=jax.ShapeDtypeStruct((K, D), jnp.float32),
    )(vals.reshape(K // BS, BS, 1), gathered)

    return scaled.T
